# baseline (speedup 1.0000x reference)
import functools

import jax
import jax.numpy as jnp
from jax import lax
from jax.experimental import pallas as pl
from jax.experimental.pallas import tpu as pltpu
from jax.experimental.pallas import tpu_sc as plsc

A = 3
N = 2048
Q = 16
ROWS = A * N
NW = 32
RPW = ROWS // NW
CH = 12
NCH = RPW // CH

_OFF_000 = 0
_OFF_011 = 1
_OFF_101 = 10
_OFF_110 = 19
_OFF_111 = 28
_OFF_112 = 55
CG_LEN = 112

COLS = (512, 768, 1536, 1280)


def _build_phases():
    phases = []

    w_defs, comps = [], []
    w_defs.append([(_OFF_000, 0)])
    comps.append((0, [(0, 0)]))
    entries = []
    for j in range(3):
        w_defs.append([(_OFF_110 + i * 3 + j, 1 + i) for i in range(3)])
        entries.append((len(w_defs) - 1, 1 + j))
    comps.append((256, entries))
    phases.append((w_defs, comps))

    w_defs, comps = [], []
    for M in range(3):
        entries = []
        for j in range(3):
            w_defs.append([(_OFF_111 + (i * 3 + j) * 3 + M, 1 + i) for i in range(3)])
            entries.append((len(w_defs) - 1, 1 + j))
        comps.append((M * 256, entries))
    phases.append((w_defs, comps))

    w_defs, comps = [], []
    for M in range(3):
        entries = []
        for j in range(3):
            w_defs.append([(_OFF_011 + j * 3 + M, 0)])
            entries.append((len(w_defs) - 1, 1 + j))
        comps.append((M * 512, entries))
    for M in range(3):
        w_defs.append([(_OFF_101 + i * 3 + M, 1 + i) for i in range(3)])
        comps.append((M * 512 + 256, [(len(w_defs) - 1, 0)]))
    phases.append((w_defs, comps))

    w_defs, comps = [], []
    for M in range(5):
        entries = []
        for j in range(3):
            w_defs.append([(_OFF_112 + (i * 3 + j) * 5 + M, 1 + i) for i in range(3)])
            entries.append((len(w_defs) - 1, 1 + j))
        comps.append((M * 256, entries))
    phases.append((w_defs, comps))

    return phases


_PHASES = _build_phases()
_TAB = []
_NCOMP = 0
for _pid, (_wd, _cs) in enumerate(_PHASES):
    for _cid in range(len(_cs)):
        _TAB.append((_pid, _cid))
        _NCOMP += 1
assert _NCOMP == 16


def _sc_body(xin_hbm, cg_hbm, o01_hbm, o1m1_hbm, o11_hbm, o21_hbm,
             in_v, cg_v, idx_v, ob0, ob1, ob2, ob3,
             sem_in0, sem_in1,
             so00, so01, so02, so03, so10, so11, so12, so13):
    wid = lax.axis_index("s") * 2 + lax.axis_index("c")
    row0 = wid * RPW

    pltpu.sync_copy(cg_hbm, cg_v)
    cgvecs = [cg_v[pl.ds(16 * k, 16)] for k in range(CG_LEN // 16)]

    def cgs(i):
        return cgvecs[i // 16][i % 16]

    colbase = lax.iota(jnp.int32, 16) * 16

    t = 0
    for pid, (w_defs, comps) in enumerate(_PHASES):
        for col, entries in comps:
            for q2 in range(Q):
                idx_v[t, :] = colbase + (col + q2)
                t += 1

    out_bufs = (ob0, ob1, ob2, ob3)
    out_hbms = (o01_hbm, o1m1_hbm, o11_hbm, o21_hbm)
    sem_in = (sem_in0, sem_in1)
    sem_out = ((so00, so01, so02, so03), (so10, so11, so12, so13))

    def start_in(ch, b):
        pltpu.async_copy(xin_hbm.at[pl.ds(row0 + ch * CH, CH)],
                         in_v.at[b], sem_in[b])

    def wait_in(b):
        pltpu.make_async_copy(xin_hbm.at[pl.ds(0, CH)],
                              in_v.at[b], sem_in[b]).wait()

    def start_out(ch, b):
        base = row0 + ch * CH
        for k in range(4):
            pltpu.async_copy(out_bufs[k].at[b],
                             out_hbms[k].at[pl.ds(base, CH)], sem_out[b][k])

    def wait_out(b):
        for k in range(4):
            pltpu.make_async_copy(out_bufs[k].at[b],
                                  out_hbms[k].at[pl.ds(0, CH)],
                                  sem_out[b][k]).wait()

    def compute_chunk(b):
        bsplat = jnp.full((16,), b, jnp.int32)

        @pl.loop(0, CH)
        def _row(r):
            avec = [in_v[b, r, pl.ds(16 * c, 16)] for c in range(4)]
            bvec = [in_v[b, r, pl.ds(64 + 16 * c, 16)] for c in range(4)]
            rsplat = jnp.full((16,), r, jnp.int32)

            ci = 0
            for pid, (w_defs, comps) in enumerate(_PHASES):
                wvecs = []
                for terms in w_defs:
                    acc = None
                    for cg_idx, a_comp in terms:
                        term = cgs(cg_idx) * avec[a_comp]
                        acc = term if acc is None else acc + term
                    wvecs.append(acc)
                buf = out_bufs[pid]
                for col, entries in comps:
                    for q2 in range(Q):
                        acc = None
                        for w_idx, b_comp in entries:
                            term = wvecs[w_idx] * bvec[b_comp][q2]
                            acc = term if acc is None else acc + term
                        cidx = idx_v[ci * 16 + q2, :]
                        plsc.store_scatter(buf, [bsplat, rsplat, cidx], acc)
                    ci += 1

    start_in(0, 0)
    start_in(1, 1)

    @pl.loop(0, NCH // 2)
    def _pair(p):
        for b in range(2):
            ch = p * 2 + b
            wait_in(b)

            @pl.when(p > 0)
            def _():
                wait_out(b)

            compute_chunk(b)
            start_out(ch, b)

            @pl.when(ch + 2 < NCH)
            def _():
                start_in(ch + 2, b)

    wait_out(0)
    wait_out(1)


@jax.jit
def _run(xin, cgflat):
    mesh = plsc.VectorSubcoreMesh(core_axis_name="c", subcore_axis_name="s")
    f = pl.kernel(
        _sc_body,
        out_type=tuple(
            jax.ShapeDtypeStruct((ROWS, w), jnp.float32) for w in COLS
        ),
        mesh=mesh,
        compiler_params=pltpu.CompilerParams(
            use_tc_tiling_on_sc=False, needs_layout_passes=False),
        scratch_types=[
            pltpu.VMEM((2, CH, 128), jnp.float32),
            pltpu.VMEM((CG_LEN,), jnp.float32),
            pltpu.VMEM((16 * Q, 16), jnp.int32),
            pltpu.VMEM((2, CH, COLS[0]), jnp.float32),
            pltpu.VMEM((2, CH, COLS[1]), jnp.float32),
            pltpu.VMEM((2, CH, COLS[2]), jnp.float32),
            pltpu.VMEM((2, CH, COLS[3]), jnp.float32),
        ] + [pltpu.SemaphoreType.DMA] * 10,
    )
    return f(xin, cgflat)


def kernel(x1_l0, x1_l1, x2_l0, x2_l1, cg_0_0_0, cg_0_1_1, cg_1_0_1,
           cg_1_1_0, cg_1_1_1, cg_1_1_2, sel):
    del sel
    xin = jnp.concatenate([
        x1_l0.reshape(ROWS, Q),
        x1_l1.reshape(ROWS, 3 * Q),
        x2_l0.reshape(ROWS, Q),
        x2_l1.reshape(ROWS, 3 * Q),
    ], axis=1)
    cgflat = jnp.concatenate([
        cg_0_0_0.ravel(), cg_0_1_1.ravel(), cg_1_0_1.ravel(),
        cg_1_1_0.ravel(), cg_1_1_1.ravel(), cg_1_1_2.ravel(),
        jnp.zeros((CG_LEN - 100,), jnp.float32),
    ])
    o01, o1m1, o11, o21 = _run(xin, cgflat)
    return (
        o01.reshape(A, N, 1, 512),
        o1m1.reshape(A, N, 3, 256),
        o11.reshape(A, N, 3, 512),
        o21.reshape(A, N, 5, 256),
    )


# speedup vs baseline: 2.7666x; 2.7666x over previous
import functools

import jax
import jax.numpy as jnp
from jax import lax
from jax.experimental import pallas as pl
from jax.experimental.pallas import tpu as pltpu
from jax.experimental.pallas import tpu_sc as plsc

A = 3
N = 2048
Q = 16
ROWS = A * N
NW = 32
RPW = ROWS // NW
CH = 8
NCH = RPW // CH

_OFF_000 = 0
_OFF_011 = 1
_OFF_101 = 10
_OFF_110 = 19
_OFF_111 = 28
_OFF_112 = 55
CG_LEN = 112

OUT_DIMS = ((1, 512), (3, 256), (3, 512), (5, 256))
NTILE = N // 8


def _build_plan():
    w_terms, comps = [], []

    def new_w(terms):
        w_terms.append(terms)
        return len(w_terms) - 1

    w = new_w([(_OFF_000, 0)])
    comps.append((0, 0, 0, [(w, 0)]))
    entries = []
    for j in range(3):
        w = new_w([(_OFF_110 + i * 3 + j, 1 + i) for i in range(3)])
        entries.append((w, 1 + j))
    comps.append((0, 0, 256, entries))
    for M in range(3):
        entries = []
        for j in range(3):
            w = new_w([(_OFF_111 + (i * 3 + j) * 3 + M, 1 + i) for i in range(3)])
            entries.append((w, 1 + j))
        comps.append((1, M, 0, entries))
    for M in range(3):
        entries = []
        for j in range(3):
            w = new_w([(_OFF_011 + j * 3 + M, 0)])
            entries.append((w, 1 + j))
        comps.append((2, M, 0, entries))
    for M in range(3):
        w = new_w([(_OFF_101 + i * 3 + M, 1 + i) for i in range(3)])
        comps.append((2, M, 256, [(w, 0)]))
    for M in range(5):
        entries = []
        for j in range(3):
            w = new_w([(_OFF_112 + (i * 3 + j) * 5 + M, 1 + i) for i in range(3)])
            entries.append((w, 1 + j))
        comps.append((3, M, 0, entries))

    return w_terms, comps


_W_TERMS, _COMPS = _build_plan()

_PT = [p // 128 for _, p in OUT_DIMS]
_KEYSTRIDE = [m * pt * 1024 for (m, _), pt in zip(OUT_DIMS, _PT)]


def _sc_body(xin_hbm, cg_hbm, o01_hbm, o1m1_hbm, o11_hbm, o21_hbm,
             in_v, cg_v, b01, b1m1, b11, b21, sem_out):
    wid = lax.axis_index("s") * 2 + lax.axis_index("c")
    row0 = wid * RPW

    pltpu.sync_copy(cg_hbm, cg_v)
    cgvecs = [cg_v[pl.ds(16 * k, 16)] for k in range(CG_LEN // 16)]

    def cgs(i):
        return cgvecs[i // 16][i % 16]

    iota = lax.iota(jnp.int32, 16)
    colbase = iota * 16
    tilebase = (iota // 8) * 1024 + (iota % 8) * 16

    out_bufs = (b01, b1m1, b11, b21)
    out_hbms = (o01_hbm, o1m1_hbm, o11_hbm, o21_hbm)

    @pl.loop(0, NCH)
    def _chunk(ch):
        base = row0 + ch * CH
        pltpu.sync_copy(xin_hbm.at[pl.ds(base, CH)], in_v)

        @pl.loop(0, CH)
        def _row(r):
            avec = [in_v[r, pl.ds(16 * c, 16)] for c in range(4)]
            bvec = [in_v[r, pl.ds(64 + 16 * c, 16)] for c in range(4)]

            wvecs = []
            for terms in _W_TERMS:
                acc = None
                for cg_idx, a_comp in terms:
                    term = cgs(cg_idx) * avec[a_comp]
                    acc = term if acc is None else acc + term
                wvecs.append(acc)

            vb01 = colbase + r * 512
            vbt = tilebase + r * 128

            for q2 in range(Q):
                bs = [bvec[c][q2] for c in range(4)]
                for buf_id, m_idx, p_off, entries in _COMPS:
                    acc = None
                    for w_idx, b_comp in entries:
                        term = wvecs[w_idx] * bs[b_comp]
                        acc = term if acc is None else acc + term
                    if buf_id == 0:
                        fvec = vb01 + (p_off + q2)
                    else:
                        fvec = vbt + (m_idx * _PT[buf_id] * 1024
                                      + (p_off // 128) * 1024 + q2)
                    plsc.store_scatter(out_bufs[buf_id], [fvec], acc)

        a_sp = base // 2048
        off = (base % 2048) * 128
        copies = [(b01, o01_hbm.at[pl.ds(base * 512, CH * 512)])]
        for k in (1, 2, 3):
            m = OUT_DIMS[k][0]
            pt = _PT[k]
            for m_i in range(m):
                copies.append((
                    out_bufs[k].at[pl.ds(m_i * pt * 1024, pt * 1024)],
                    out_hbms[k].at[a_sp, m_i, pl.ds(off * pt, pt * 1024)],
                ))
        descs = [pltpu.async_copy(src, dst, sem_out) for src, dst in copies]
        for d in descs:
            d.wait()


@jax.jit
def _run(xin, cgflat):
    mesh = plsc.VectorSubcoreMesh(core_axis_name="c", subcore_axis_name="s")
    f = pl.kernel(
        _sc_body,
        out_type=(
            jax.ShapeDtypeStruct((ROWS * 512,), jnp.float32),
            jax.ShapeDtypeStruct((A, 3, NTILE * 2 * 1024), jnp.float32),
            jax.ShapeDtypeStruct((A, 3, NTILE * 4 * 1024), jnp.float32),
            jax.ShapeDtypeStruct((A, 5, NTILE * 2 * 1024), jnp.float32),
        ),
        mesh=mesh,
        compiler_params=pltpu.CompilerParams(
            use_tc_tiling_on_sc=False, needs_layout_passes=False,
            disable_bounds_checks=True),
        scratch_types=[
            pltpu.VMEM((CH, 128), jnp.float32),
            pltpu.VMEM((CG_LEN,), jnp.float32),
            pltpu.VMEM((CH * 512,), jnp.float32),
            pltpu.VMEM((3 * 2 * 1024,), jnp.float32),
            pltpu.VMEM((3 * 4 * 1024,), jnp.float32),
            pltpu.VMEM((5 * 2 * 1024,), jnp.float32),
            pltpu.SemaphoreType.DMA,
        ],
    )
    return f(xin, cgflat)


def _untile(o, m, p):
    pt = p // 128
    return (o.reshape(A, m, NTILE, pt, 8, 128)
             .transpose(0, 2, 4, 1, 3, 5)
             .reshape(A, N, m, p))


def kernel(x1_l0, x1_l1, x2_l0, x2_l1, cg_0_0_0, cg_0_1_1, cg_1_0_1,
           cg_1_1_0, cg_1_1_1, cg_1_1_2, sel):
    del sel
    xin = jnp.concatenate([
        x1_l0.reshape(ROWS, Q),
        x1_l1.reshape(ROWS, 3 * Q),
        x2_l0.reshape(ROWS, Q),
        x2_l1.reshape(ROWS, 3 * Q),
    ], axis=1)
    cgflat = jnp.concatenate([
        cg_0_0_0.ravel(), cg_0_1_1.ravel(), cg_1_0_1.ravel(),
        cg_1_1_0.ravel(), cg_1_1_1.ravel(), cg_1_1_2.ravel(),
        jnp.zeros((CG_LEN - 100,), jnp.float32),
    ])
    o01, o1m1, o11, o21 = _run(xin, cgflat)
    return (
        o01.reshape(A, N, 1, 512),
        _untile(o1m1, 3, 256),
        _untile(o11, 3, 512),
        _untile(o21, 5, 256),
    )


# speedup vs baseline: 3.3775x; 1.2208x over previous
import functools

import jax
import jax.numpy as jnp
from jax import lax
from jax.experimental import pallas as pl
from jax.experimental.pallas import tpu as pltpu
from jax.experimental.pallas import tpu_sc as plsc

A = 3
N = 2048
Q = 16
ROWS = A * N
NW = 32
RPW = ROWS // NW
CH = 8
NCH = RPW // CH

_OFF_000 = 0
_OFF_011 = 1
_OFF_101 = 10
_OFF_110 = 19
_OFF_111 = 28
_OFF_112 = 55
CG_LEN = 112

OUT_DIMS = ((1, 512), (3, 256), (3, 512), (5, 256))
NTILE = N // 8
_PT = [p // 128 for _, p in OUT_DIMS]
_PHASE_ORDER = (2, 3, 1, 0)


def _build_phases():
    phases = {}

    w_defs, comps = [], []
    w_defs.append([(_OFF_000, 0)])
    comps.append((0, 0, [(0, 0)]))
    entries = []
    for j in range(3):
        w_defs.append([(_OFF_110 + i * 3 + j, 1 + i) for i in range(3)])
        entries.append((len(w_defs) - 1, 1 + j))
    comps.append((0, 256, entries))
    phases[0] = (w_defs, comps)

    w_defs, comps = [], []
    for M in range(3):
        entries = []
        for j in range(3):
            w_defs.append([(_OFF_111 + (i * 3 + j) * 3 + M, 1 + i) for i in range(3)])
            entries.append((len(w_defs) - 1, 1 + j))
        comps.append((M, 0, entries))
    phases[1] = (w_defs, comps)

    w_defs, comps = [], []
    for M in range(3):
        entries = []
        for j in range(3):
            w_defs.append([(_OFF_011 + j * 3 + M, 0)])
            entries.append((len(w_defs) - 1, 1 + j))
        comps.append((M, 0, entries))
    for M in range(3):
        w_defs.append([(_OFF_101 + i * 3 + M, 1 + i) for i in range(3)])
        comps.append((M, 256, [(len(w_defs) - 1, 0)]))
    phases[2] = (w_defs, comps)

    w_defs, comps = [], []
    for M in range(5):
        entries = []
        for j in range(3):
            w_defs.append([(_OFF_112 + (i * 3 + j) * 5 + M, 1 + i) for i in range(3)])
            entries.append((len(w_defs) - 1, 1 + j))
        comps.append((M, 0, entries))
    phases[3] = (w_defs, comps)

    return phases


_PHASES = _build_phases()


def _sc_body(xin_hbm, cg_hbm, o01_hbm, o1m1_hbm, o11_hbm, o21_hbm,
             in_v, cg_v, b01, b1m1, b11, b21,
             sem_in0, sem_in1, sp0, sp1, sp2, sp3):
    wid = lax.axis_index("s") * 2 + lax.axis_index("c")
    row0 = wid * RPW

    pltpu.sync_copy(cg_hbm, cg_v)
    cgvecs = [cg_v[pl.ds(16 * k, 16)] for k in range(CG_LEN // 16)]

    def cgs(i):
        return cgvecs[i // 16][i % 16]

    iota = lax.iota(jnp.int32, 16)
    colbase = iota * 16
    tilebase = (iota // 8) * 1024 + (iota % 8) * 16

    out_bufs = (b01, b1m1, b11, b21)
    out_hbms = (o01_hbm, o1m1_hbm, o11_hbm, o21_hbm)
    sem_in = (sem_in0, sem_in1)
    sem_ph = (sp0, sp1, sp2, sp3)

    def start_in(ch, b):
        pltpu.async_copy(xin_hbm.at[pl.ds(row0 + ch * CH, CH)],
                         in_v.at[b], sem_in[b])

    def wait_in(b):
        pltpu.make_async_copy(xin_hbm.at[pl.ds(0, CH)],
                              in_v.at[b], sem_in[b]).wait()

    def phase_copies(ch, key):
        base = row0 + ch * CH
        if key == 0:
            return [(b01, o01_hbm.at[pl.ds(base * 512, CH * 512)])]
        a_sp = base // 2048
        off = (base % 2048) * 128
        m = OUT_DIMS[key][0]
        pt = _PT[key]
        return [
            (out_bufs[key].at[pl.ds(m_i * pt * 1024, pt * 1024)],
             out_hbms[key].at[a_sp, m_i, pl.ds(off * pt, pt * 1024)])
            for m_i in range(m)
        ]

    def start_phase_out(ch, key):
        for src, dst in phase_copies(ch, key):
            pltpu.async_copy(src, dst, sem_ph[key])

    def wait_phase_out(key):
        for src, dst in phase_copies(0, key):
            pltpu.make_async_copy(src, dst, sem_ph[key]).wait()

    def compute_phase(b, key):
        w_defs, comps = _PHASES[key]
        buf = out_bufs[key]
        pt1024 = _PT[key] * 1024

        @pl.loop(0, CH)
        def _row(r):
            avec = [in_v[b, r, pl.ds(16 * c, 16)] for c in range(4)]
            bvec = [in_v[b, r, pl.ds(64 + 16 * c, 16)] for c in range(4)]

            wvecs = []
            for terms in w_defs:
                acc = None
                for cg_idx, a_comp in terms:
                    term = cgs(cg_idx) * avec[a_comp]
                    acc = term if acc is None else acc + term
                wvecs.append(acc)

            vb = (colbase + r * 512) if key == 0 else (tilebase + r * 128)
            for q2 in range(Q):
                bs = [bvec[c][q2] for c in range(4)]
                for m_idx, p_off, entries in comps:
                    acc = None
                    for w_idx, b_comp in entries:
                        term = wvecs[w_idx] * bs[b_comp]
                        acc = term if acc is None else acc + term
                    if key == 0:
                        fvec = vb + (p_off + q2)
                    else:
                        fvec = vb + (m_idx * pt1024
                                     + (p_off // 128) * 1024 + q2)
                    plsc.store_scatter(buf, [fvec], acc)

    start_in(0, 0)

    @pl.loop(0, NCH // 2)
    def _pair(p):
        for b in range(2):
            ch = p * 2 + b
            wait_in(b)

            @pl.when(ch + 1 < NCH)
            def _():
                start_in(ch + 1, 1 - b)

            for key in _PHASE_ORDER:
                if b == 0:
                    @pl.when(p > 0)
                    def _(key=key):
                        wait_phase_out(key)
                else:
                    wait_phase_out(key)
                compute_phase(b, key)
                start_phase_out(ch, key)

    for key in _PHASE_ORDER:
        wait_phase_out(key)


@jax.jit
def _run(xin, cgflat):
    mesh = plsc.VectorSubcoreMesh(core_axis_name="c", subcore_axis_name="s")
    f = pl.kernel(
        _sc_body,
        out_type=(
            jax.ShapeDtypeStruct((ROWS * 512,), jnp.float32),
            jax.ShapeDtypeStruct((A, 3, NTILE * 2 * 1024), jnp.float32),
            jax.ShapeDtypeStruct((A, 3, NTILE * 4 * 1024), jnp.float32),
            jax.ShapeDtypeStruct((A, 5, NTILE * 2 * 1024), jnp.float32),
        ),
        mesh=mesh,
        compiler_params=pltpu.CompilerParams(
            use_tc_tiling_on_sc=False, needs_layout_passes=False,
            disable_bounds_checks=True),
        scratch_types=[
            pltpu.VMEM((2, CH, 128), jnp.float32),
            pltpu.VMEM((CG_LEN,), jnp.float32),
            pltpu.VMEM((CH * 512,), jnp.float32),
            pltpu.VMEM((3 * 2 * 1024,), jnp.float32),
            pltpu.VMEM((3 * 4 * 1024,), jnp.float32),
            pltpu.VMEM((5 * 2 * 1024,), jnp.float32),
        ] + [pltpu.SemaphoreType.DMA] * 6,
    )
    return f(xin, cgflat)


def _untile(o, m, p):
    pt = p // 128
    return (o.reshape(A, m, NTILE, pt, 8, 128)
             .transpose(0, 2, 4, 1, 3, 5)
             .reshape(A, N, m, p))


def kernel(x1_l0, x1_l1, x2_l0, x2_l1, cg_0_0_0, cg_0_1_1, cg_1_0_1,
           cg_1_1_0, cg_1_1_1, cg_1_1_2, sel):
    del sel
    xin = jnp.concatenate([
        x1_l0.reshape(ROWS, Q),
        x1_l1.reshape(ROWS, 3 * Q),
        x2_l0.reshape(ROWS, Q),
        x2_l1.reshape(ROWS, 3 * Q),
    ], axis=1)
    cgflat = jnp.concatenate([
        cg_0_0_0.ravel(), cg_0_1_1.ravel(), cg_1_0_1.ravel(),
        cg_1_1_0.ravel(), cg_1_1_1.ravel(), cg_1_1_2.ravel(),
        jnp.zeros((CG_LEN - 100,), jnp.float32),
    ])
    o01, o1m1, o11, o21 = _run(xin, cgflat)
    return (
        o01.reshape(A, N, 1, 512),
        _untile(o1m1, 3, 256),
        _untile(o11, 3, 512),
        _untile(o21, 5, 256),
    )


# speedup vs baseline: 7.8399x; 2.3212x over previous
import functools

import jax
import jax.numpy as jnp
from jax import lax
from jax.experimental import pallas as pl
from jax.experimental.pallas import tpu as pltpu
from jax.experimental.pallas import tpu_sc as plsc

A = 3
N = 2048
Q = 16
ROWS = A * N
NW = 32
RPW = ROWS // NW
CH = 8
NCH = RPW // CH

_OFF_000 = 0
_OFF_011 = 1
_OFF_101 = 10
_OFF_110 = 19
_OFF_111 = 28
_OFF_112 = 55
CG_LEN = 112

NTILE = N // 8
BN = 256
NB = N // BN


def _build_sc_phases():
    phases = {}

    w_defs, comps = [], []
    w_defs.append([(_OFF_000, 0)])
    comps.append((0, 0, [(0, 0)]))
    entries = []
    for j in range(3):
        w_defs.append([(_OFF_110 + i * 3 + j, 1 + i) for i in range(3)])
        entries.append((len(w_defs) - 1, 1 + j))
    comps.append((0, 256, entries))
    phases[0] = (w_defs, comps)

    w_defs, comps = [], []
    for M in range(3):
        entries = []
        for j in range(3):
            w_defs.append([(_OFF_111 + (i * 3 + j) * 3 + M, 1 + i) for i in range(3)])
            entries.append((len(w_defs) - 1, 1 + j))
        comps.append((M, 0, entries))
    phases[1] = (w_defs, comps)

    return phases


_SC_PHASES = _build_sc_phases()
_SC_ORDER = (1, 0)
_SC_PT = (4, 2)


def _sc_body(xin_hbm, cg_hbm, o01_hbm, o1m1_hbm,
             in_v, cg_v, b01, b1m1,
             sem_in0, sem_in1, sp0, sp1):
    wid = lax.axis_index("s") * 2 + lax.axis_index("c")
    row0 = wid * RPW

    pltpu.sync_copy(cg_hbm, cg_v)
    cgvecs = [cg_v[pl.ds(16 * k, 16)] for k in range(CG_LEN // 16)]

    def cgs(i):
        return cgvecs[i // 16][i % 16]

    iota = lax.iota(jnp.int32, 16)
    colbase = iota * 16
    tilebase = (iota // 8) * 1024 + (iota % 8) * 16

    out_bufs = (b01, b1m1)
    sem_in = (sem_in0, sem_in1)
    sem_ph = (sp0, sp1)

    def start_in(ch, b):
        pltpu.async_copy(xin_hbm.at[pl.ds(row0 + ch * CH, CH)],
                         in_v.at[b], sem_in[b])

    def wait_in(b):
        pltpu.make_async_copy(xin_hbm.at[pl.ds(0, CH)],
                              in_v.at[b], sem_in[b]).wait()

    def phase_copies(ch, key):
        base = row0 + ch * CH
        if key == 0:
            return [(b01, o01_hbm.at[pl.ds(base * 512, CH * 512)])]
        a_sp = base // 2048
        off = (base % 2048) * 128
        return [
            (b1m1.at[pl.ds(m_i * 2048, 2048)],
             o1m1_hbm.at[a_sp, m_i, pl.ds(off * 2, 2048)])
            for m_i in range(3)
        ]

    def start_phase_out(ch, key):
        for src, dst in phase_copies(ch, key):
            pltpu.async_copy(src, dst, sem_ph[key])

    def wait_phase_out(key):
        for src, dst in phase_copies(0, key):
            pltpu.make_async_copy(src, dst, sem_ph[key]).wait()

    def compute_phase(b, key):
        w_defs, comps = _SC_PHASES[key]
        buf = out_bufs[key]

        @pl.loop(0, CH)
        def _row(r):
            avec = [in_v[b, r, pl.ds(16 * c, 16)] for c in range(4)]
            bvec = [in_v[b, r, pl.ds(64 + 16 * c, 16)] for c in range(4)]

            wvecs = []
            for terms in w_defs:
                acc = None
                for cg_idx, a_comp in terms:
                    term = cgs(cg_idx) * avec[a_comp]
                    acc = term if acc is None else acc + term
                wvecs.append(acc)

            vb = (colbase + r * 512) if key == 0 else (tilebase + r * 128)
            for q2 in range(Q):
                bs = [bvec[c][q2] for c in range(4)]
                for m_idx, p_off, entries in comps:
                    acc = None
                    for w_idx, b_comp in entries:
                        term = wvecs[w_idx] * bs[b_comp]
                        acc = term if acc is None else acc + term
                    if key == 0:
                        fvec = vb + (p_off + q2)
                    else:
                        fvec = vb + (m_idx * 2048 + q2)
                    plsc.store_scatter(buf, [fvec], acc)

    start_in(0, 0)

    @pl.loop(0, NCH // 2)
    def _pair(p):
        for b in range(2):
            ch = p * 2 + b
            wait_in(b)

            @pl.when(ch + 1 < NCH)
            def _():
                start_in(ch + 1, 1 - b)

            for key in _SC_ORDER:
                if b == 0:
                    @pl.when(p > 0)
                    def _(key=key):
                        wait_phase_out(key)
                else:
                    wait_phase_out(key)
                compute_phase(b, key)
                start_phase_out(ch, key)

    for key in _SC_ORDER:
        wait_phase_out(key)


def _tc_body(cg_ref, xin_ref, o11_ref, o21_ref):
    x = xin_ref[0]
    qi = lax.broadcasted_iota(jnp.int32, (Q, 256), 0)
    pi = lax.broadcasted_iota(jnp.int32, (Q, 256), 1)
    Rm = (pi // Q == qi).astype(jnp.float32)
    Tm = (pi % Q == qi).astype(jnp.float32)

    x1l1 = x[:, 16:64]
    x2l0 = x[:, 64:80]
    x2l1 = x[:, 80:128]
    x1l0 = x[:, 0:16]

    def mm(a, w):
        return jax.lax.dot_general(
            a, w, (((1,), (0,)), ((), ())),
            preferred_element_type=jnp.float32)

    A0 = mm(x1l0, Rm)
    B0 = mm(x2l0, Tm)
    B1 = [mm(x2l1[:, 16 * j:16 * (j + 1)], Tm) for j in range(3)]

    def wstack(coeffs, base):
        return jnp.concatenate([c * base for c in coeffs], axis=0)

    for M in range(5):
        acc = None
        for j in range(3):
            w = wstack([cg_ref[_OFF_112 + (i * 3 + j) * 5 + M]
                        for i in range(3)], Rm)
            t = mm(x1l1, w) * B1[j]
            acc = t if acc is None else acc + t
        o21_ref[0, M] = acc

    for M in range(3):
        wb = wstack([cg_ref[_OFF_011 + j * 3 + M] for j in range(3)], Tm)
        left = A0 * mm(x2l1, wb)
        wa = wstack([cg_ref[_OFF_101 + i * 3 + M] for i in range(3)], Rm)
        right = mm(x1l1, wa) * B0
        o11_ref[0, M] = jnp.concatenate([left, right], axis=1)


@jax.jit
def _run(xin, cgflat):
    mesh = plsc.VectorSubcoreMesh(core_axis_name="c", subcore_axis_name="s")
    sc = pl.kernel(
        _sc_body,
        out_type=(
            jax.ShapeDtypeStruct((ROWS * 512,), jnp.float32),
            jax.ShapeDtypeStruct((A, 3, NTILE * 2 * 1024), jnp.float32),
        ),
        mesh=mesh,
        compiler_params=pltpu.CompilerParams(
            use_tc_tiling_on_sc=False, needs_layout_passes=False,
            disable_bounds_checks=True),
        scratch_types=[
            pltpu.VMEM((2, CH, 128), jnp.float32),
            pltpu.VMEM((CG_LEN,), jnp.float32),
            pltpu.VMEM((CH * 512,), jnp.float32),
            pltpu.VMEM((3 * 2 * 1024,), jnp.float32),
        ] + [pltpu.SemaphoreType.DMA] * 4,
    )
    o01, o1m1 = sc(xin, cgflat)

    xin3 = xin.reshape(A, N, 128)
    o11t, o21t = pl.pallas_call(
        _tc_body,
        grid=(A, NB),
        in_specs=[
            pl.BlockSpec(memory_space=pltpu.SMEM),
            pl.BlockSpec((1, BN, 128), lambda a, nb: (a, nb, 0)),
        ],
        out_specs=[
            pl.BlockSpec((1, 3, BN, 512), lambda a, nb: (a, 0, nb, 0)),
            pl.BlockSpec((1, 5, BN, 256), lambda a, nb: (a, 0, nb, 0)),
        ],
        out_shape=[
            jax.ShapeDtypeStruct((A, 3, N, 512), jnp.float32),
            jax.ShapeDtypeStruct((A, 5, N, 256), jnp.float32),
        ],
    )(cgflat, xin3)
    return o01, o1m1, o11t, o21t


def _untile(o, m, p):
    pt = p // 128
    return (o.reshape(A, m, NTILE, pt, 8, 128)
             .transpose(0, 2, 4, 1, 3, 5)
             .reshape(A, N, m, p))


def kernel(x1_l0, x1_l1, x2_l0, x2_l1, cg_0_0_0, cg_0_1_1, cg_1_0_1,
           cg_1_1_0, cg_1_1_1, cg_1_1_2, sel):
    del sel
    xin = jnp.concatenate([
        x1_l0.reshape(ROWS, Q),
        x1_l1.reshape(ROWS, 3 * Q),
        x2_l0.reshape(ROWS, Q),
        x2_l1.reshape(ROWS, 3 * Q),
    ], axis=1)
    cgflat = jnp.concatenate([
        cg_0_0_0.ravel(), cg_0_1_1.ravel(), cg_1_0_1.ravel(),
        cg_1_1_0.ravel(), cg_1_1_1.ravel(), cg_1_1_2.ravel(),
        jnp.zeros((CG_LEN - 100,), jnp.float32),
    ])
    o01, o1m1, o11t, o21t = _run(xin, cgflat)
    return (
        o01.reshape(A, N, 1, 512),
        _untile(o1m1, 3, 256),
        o11t.transpose(0, 2, 1, 3),
        o21t.transpose(0, 2, 1, 3),
    )
